# baseline (speedup 1.0000x reference)
import jax
import jax.numpy as jnp
from jax.experimental import pallas as pl

N_NODES = 10000
N_EDGES = 320000
N_SUP = 2048


def _loss_body(pos_ref, neg_ref, out_ref):
    pos = pos_ref[...]
    neg = neg_ref[...]
    loss = jnp.mean(jax.nn.softplus(-pos)) + jnp.mean(jax.nn.softplus(neg))
    out_ref[...] = jnp.broadcast_to(loss, (1, 128))


def kernel(x, edge_attr, W_emb, W_edge, W_gcn, b_gcn, edge_index,
           pos_sup_edge_index, neg_sup_edge_index):
    h = x @ W_emb
    src = edge_index[0]
    dst = edge_index[1]
    deg = jax.ops.segment_sum(jnp.ones((N_EDGES,), dtype=jnp.float32), dst,
                              num_segments=N_NODES)
    inv_sqrt_deg = 1.0 / jnp.sqrt(jnp.clip(deg, 1.0, None))
    norm = inv_sqrt_deg[src] * inv_sqrt_deg[dst]
    aggH = jax.ops.segment_sum(jnp.take(h, src, axis=0) * norm[:, None], dst,
                               num_segments=N_NODES)
    agg16 = jax.ops.segment_sum(edge_attr * norm[:, None], dst,
                                num_segments=N_NODES)
    agg = aggH + agg16 @ W_edge
    node_emb = jax.nn.relu(agg @ W_gcn + b_gcn)
    pos_score = jnp.sum(jnp.take(node_emb, pos_sup_edge_index[0], axis=0) *
                        jnp.take(node_emb, pos_sup_edge_index[1], axis=0), axis=1)
    neg_score = jnp.sum(jnp.take(node_emb, neg_sup_edge_index[0], axis=0) *
                        jnp.take(node_emb, neg_sup_edge_index[1], axis=0), axis=1)
    loss = pl.pallas_call(
        _loss_body,
        out_shape=jax.ShapeDtypeStruct((1, 128), jnp.float32),
    )(pos_score, neg_score)
    return loss[0, 0]


# speedup vs baseline: 6.6544x; 6.6544x over previous
import functools

import jax
import jax.numpy as jnp
from jax import lax
from jax.experimental import pallas as pl
from jax.experimental.pallas import tpu as pltpu
from jax.experimental.pallas import tpu_sc as plsc

N_NODES = 10000
N_EDGES = 320000
D = 128
DE = 16
N_SUP = 2048

NC = 2
NS = 16
NT = NC * NS
CHUNK = 128
NCH = 79
EPT = NCH * CHUNK
NE_PAD = NT * EPT
ACC = 10112
ACCD = 10240
ZROWS = ACC // NS
SUP_T = N_SUP // NT

_f32 = jnp.float32
_i32 = jnp.int32


def _mesh():
    return plsc.VectorSubcoreMesh(core_axis_name="c", subcore_axis_name="s",
                                  num_cores=NC, num_subcores=NS)


def _sc_deg(dstp, ones_v, zvec):
    @functools.partial(
        pl.kernel,
        out_type=jax.ShapeDtypeStruct((NC, ACCD), _f32),
        mesh=_mesh(),
        scratch_types=[
            pltpu.VMEM((NCH, CHUNK), _i32),
            pltpu.VMEM((CHUNK,), _f32),
            pltpu.VMEM((640,), _f32),
            pltpu.VMEM((640,), _f32),
            pltpu.VMEM_SHARED((ACCD,), _f32),
        ],
    )
    def body(dst_h, ones_h, z_h, out_h, dstv, onesv, zv, degv, deg_s):
        c = lax.axis_index("c")
        s = lax.axis_index("s")
        t = c * NS + s
        pltpu.sync_copy(dst_h.at[t], dstv)
        pltpu.sync_copy(ones_h, onesv)
        pltpu.sync_copy(z_h, zv)
        pltpu.sync_copy(zv, deg_s.at[pl.ds(s * 640, 640)])
        plsc.subcore_barrier()

        def step(j, carry):
            pltpu.sync_copy(onesv, deg_s.at[dstv.at[j]], add=True)
            return carry

        lax.fori_loop(0, NCH, step, 0)
        plsc.subcore_barrier()
        pltpu.sync_copy(deg_s.at[pl.ds(s * 640, 640)], degv)
        pltpu.sync_copy(degv, out_h.at[c, pl.ds(s * 640, 640)])

    return body(dstp, ones_v, zvec)


def _tc_emb(x, W_emb, degp):
    def body(x_ref, w_ref, degp_ref, h2_ref, isd_ref):
        deg = degp_ref[0, :] + degp_ref[1, :]
        isd = lax.rsqrt(jnp.maximum(deg, 1.0))
        isd_ref[...] = isd
        h = jnp.dot(x_ref[...], w_ref[...], preferred_element_type=_f32)
        h2_ref[...] = h * isd[:N_NODES][:, None]

    return pl.pallas_call(
        body,
        out_shape=[
            jax.ShapeDtypeStruct((N_NODES, D), _f32),
            jax.ShapeDtypeStruct((ACCD,), _f32),
        ],
    )(x, W_emb, degp)


def _sc_aggH(h2, isd, srcp, dstp, zH):
    @functools.partial(
        pl.kernel,
        out_type=(
            jax.ShapeDtypeStruct((NC, ACC, D), _f32),
            jax.ShapeDtypeStruct((NT, NCH, CHUNK), _f32),
        ),
        mesh=_mesh(),
        scratch_types=[
            pltpu.VMEM((NCH, CHUNK), _i32),
            pltpu.VMEM((NCH, CHUNK), _i32),
            pltpu.VMEM((CHUNK, D), _f32),
            pltpu.VMEM((CHUNK,), _f32),
            pltpu.VMEM((ACCD,), _f32),
            pltpu.VMEM_SHARED((ACC, D), _f32),
            pltpu.VMEM_SHARED((ACCD,), _f32),
            pltpu.SemaphoreType.DMA,
        ],
    )
    def body(h2_h, isd_h, src_h, dst_h, zH_h, aggH_o, isdsrc_o,
             srcv, dstv, hrows, isdsv, isdv, aggH_s, isd_s, sem):
        c = lax.axis_index("c")
        s = lax.axis_index("s")
        t = c * NS + s
        pltpu.sync_copy(src_h.at[t], srcv)
        pltpu.sync_copy(dst_h.at[t], dstv)
        pltpu.sync_copy(zH_h, aggH_s.at[pl.ds(s * ZROWS, ZROWS)])

        @pl.when(s == 0)
        def _():
            pltpu.sync_copy(isd_h, isdv)
            pltpu.sync_copy(isdv, isd_s)

        plsc.subcore_barrier()

        def step(j, carry):
            pltpu.async_copy(isd_s.at[srcv.at[j]], isdsv, sem).wait()
            pltpu.sync_copy(isdsv, isdsrc_o.at[t, j])
            pltpu.async_copy(h2_h.at[srcv.at[j]], hrows, sem).wait()
            pltpu.sync_copy(hrows, aggH_s.at[dstv.at[j]], add=True)
            return carry

        lax.fori_loop(0, NCH, step, 0)
        plsc.subcore_barrier()
        pltpu.sync_copy(aggH_s.at[pl.ds(s * ZROWS, ZROWS)],
                        aggH_o.at[c, pl.ds(s * ZROWS, ZROWS)])

    return body(h2, isd, srcp, dstp, zH)


def _tc_scale(eap, isdsrc):
    G = 64
    RB = NE_PAD // G

    def body(ea_ref, s_ref, out_ref):
        out_ref[...] = ea_ref[...] * s_ref[...]

    return pl.pallas_call(
        body,
        grid=(G,),
        in_specs=[
            pl.BlockSpec((RB, DE), lambda i: (i, 0)),
            pl.BlockSpec((RB, 1), lambda i: (i, 0)),
        ],
        out_specs=pl.BlockSpec((RB, DE), lambda i: (i, 0)),
        out_shape=jax.ShapeDtypeStruct((NE_PAD, DE), _f32),
    )(eap, isdsrc.reshape(NE_PAD, 1))


def _sc_agg16(ea2p, dstp, z16):
    @functools.partial(
        pl.kernel,
        out_type=jax.ShapeDtypeStruct((NC, ACC, DE), _f32),
        mesh=_mesh(),
        scratch_types=[
            pltpu.VMEM((NCH, CHUNK), _i32),
            pltpu.VMEM((CHUNK, DE), _f32),
            pltpu.VMEM_SHARED((ACC, DE), _f32),
            pltpu.SemaphoreType.DMA,
        ],
    )
    def body(ea2_h, dst_h, z16_h, agg16_o, dstv, eav, agg16_s, sem):
        c = lax.axis_index("c")
        s = lax.axis_index("s")
        t = c * NS + s
        pltpu.sync_copy(dst_h.at[t], dstv)
        pltpu.sync_copy(z16_h, agg16_s.at[pl.ds(s * ZROWS, ZROWS)])
        plsc.subcore_barrier()
        base = t * EPT

        def step(j, carry):
            pltpu.sync_copy(ea2_h.at[pl.ds(base + j * CHUNK, CHUNK)], eav)
            pltpu.sync_copy(eav, agg16_s.at[dstv.at[j]], add=True)
            return carry

        lax.fori_loop(0, NCH, step, 0)
        plsc.subcore_barrier()
        pltpu.sync_copy(agg16_s.at[pl.ds(s * ZROWS, ZROWS)],
                        agg16_o.at[c, pl.ds(s * ZROWS, ZROWS)])

    return body(ea2p, dstp, z16)


def _tc_node(aggHp, agg16p, isd, W_edge, W_gcn, b_gcn):
    def body(aggH_ref, agg16_ref, isd_ref, we_ref, wg_ref, b_ref, out_ref):
        aggH = aggH_ref[0, :N_NODES] + aggH_ref[1, :N_NODES]
        agg16 = agg16_ref[0, :N_NODES] + agg16_ref[1, :N_NODES]
        wg = wg_ref[...]
        wec = jnp.dot(we_ref[...], wg, preferred_element_type=_f32)
        tt = (jnp.dot(aggH, wg, preferred_element_type=_f32) +
              jnp.dot(agg16, wec, preferred_element_type=_f32))
        isd = isd_ref[...][:N_NODES]
        out_ref[...] = jnp.maximum(
            tt * isd[:, None] + b_ref[...][None, :], 0.0)

    return pl.pallas_call(
        body,
        out_shape=jax.ShapeDtypeStruct((N_NODES, D), _f32),
    )(aggHp, agg16p, isd, W_edge, W_gcn, b_gcn)


def _sc_sup(node_emb, sup4):
    @functools.partial(
        pl.kernel,
        out_type=jax.ShapeDtypeStruct((4, N_SUP, D), _f32),
        mesh=_mesh(),
        scratch_types=[
            pltpu.VMEM((SUP_T,), _i32),
            pltpu.VMEM((SUP_T, D), _f32),
            pltpu.SemaphoreType.DMA,
        ],
    )
    def body(ne_h, sup_h, out_h, idxv, rows, sem):
        c = lax.axis_index("c")
        s = lax.axis_index("s")
        t = c * NS + s
        for k in range(4):
            pltpu.sync_copy(sup_h.at[k, t], idxv)
            pltpu.async_copy(ne_h.at[idxv], rows, sem).wait()
            pltpu.sync_copy(rows, out_h.at[k, pl.ds(t * SUP_T, SUP_T)])

    return body(node_emb, sup4)


def _tc_loss(rows4):
    def body(r_ref, out_ref):
        ps = jnp.sum(r_ref[0] * r_ref[1], axis=1)
        ns = jnp.sum(r_ref[2] * r_ref[3], axis=1)

        def sp(v):
            return jnp.maximum(v, 0.0) + jnp.log1p(jnp.exp(-jnp.abs(v)))

        loss = jnp.mean(sp(-ps)) + jnp.mean(sp(ns))
        out_ref[...] = jnp.broadcast_to(loss, (1, D))

    return pl.pallas_call(
        body,
        out_shape=jax.ShapeDtypeStruct((1, D), _f32),
    )(rows4)


def kernel(x, edge_attr, W_emb, W_edge, W_gcn, b_gcn, edge_index,
           pos_sup_edge_index, neg_sup_edge_index):
    src = edge_index[0]
    dst = edge_index[1]
    npad = NE_PAD - N_EDGES
    dstp = jnp.concatenate([dst, jnp.full((npad,), N_NODES, _i32)]).reshape(NT, NCH, CHUNK)
    srcp = jnp.concatenate([src, jnp.zeros((npad,), _i32)]).reshape(NT, NCH, CHUNK)
    eap = jnp.concatenate([edge_attr, jnp.zeros((npad, DE), _f32)])
    sup4 = jnp.concatenate([pos_sup_edge_index, neg_sup_edge_index]).reshape(4, NT, SUP_T)
    ones_v = jnp.ones((CHUNK,), _f32)
    zvec = jnp.zeros((640,), _f32)
    zH = jnp.zeros((ZROWS, D), _f32)
    z16 = jnp.zeros((ZROWS, DE), _f32)

    degp = _sc_deg(dstp, ones_v, zvec)
    h2, isd = _tc_emb(x, W_emb, degp)
    aggHp, isdsrc = _sc_aggH(h2, isd, srcp, dstp, zH)
    ea2p = _tc_scale(eap, isdsrc.reshape(NE_PAD))
    agg16p = _sc_agg16(ea2p, dstp, z16)
    node_emb = _tc_node(aggHp, agg16p, isd, W_edge, W_gcn, b_gcn)
    rows4 = _sc_sup(node_emb, sup4)
    loss = _tc_loss(rows4)
    return loss[0, 0]


# speedup vs baseline: 8.6577x; 1.3010x over previous
import functools

import jax
import jax.numpy as jnp
from jax import lax
from jax.experimental import pallas as pl
from jax.experimental.pallas import tpu as pltpu
from jax.experimental.pallas import tpu_sc as plsc

N_NODES = 10000
N_EDGES = 320000
D = 128
DE = 16
N_SUP = 2048

NC = 2
NS = 16
NT = NC * NS
CHUNK = 128
NCH = 79
EPT = NCH * CHUNK
NE_PAD = NT * EPT
ACC = 10112
ACCD = 10240
ZROWS = ACC // NS
SUP_T = N_SUP // NT

_f32 = jnp.float32
_i32 = jnp.int32


def _mesh():
    return plsc.VectorSubcoreMesh(core_axis_name="c", subcore_axis_name="s",
                                  num_cores=NC, num_subcores=NS)


def _sc_deg(dstp, ones_v, zvec):
    @functools.partial(
        pl.kernel,
        out_type=jax.ShapeDtypeStruct((NC, ACCD), _f32),
        mesh=_mesh(),
        scratch_types=[
            pltpu.VMEM((NCH, CHUNK), _i32),
            pltpu.VMEM((CHUNK,), _f32),
            pltpu.VMEM((640,), _f32),
            pltpu.VMEM((640,), _f32),
            pltpu.VMEM_SHARED((ACCD,), _f32),
        ],
    )
    def body(dst_h, ones_h, z_h, out_h, dstv, onesv, zv, degv, deg_s):
        c = lax.axis_index("c")
        s = lax.axis_index("s")
        t = c * NS + s
        pltpu.sync_copy(dst_h.at[t], dstv)
        pltpu.sync_copy(ones_h, onesv)
        pltpu.sync_copy(z_h, zv)
        pltpu.sync_copy(zv, deg_s.at[pl.ds(s * 640, 640)])
        plsc.subcore_barrier()

        def step(j, carry):
            pltpu.sync_copy(onesv, deg_s.at[dstv.at[j]], add=True)
            return carry

        lax.fori_loop(0, NCH, step, 0)
        plsc.subcore_barrier()
        pltpu.sync_copy(deg_s.at[pl.ds(s * 640, 640)], degv)
        pltpu.sync_copy(degv, out_h.at[c, pl.ds(s * 640, 640)])

    return body(dstp, ones_v, zvec)


def _tc_emb(x, W_emb, degp):
    def body(x_ref, w_ref, degp_ref, h2_ref, isd_ref):
        deg = degp_ref[0, :] + degp_ref[1, :]
        isd = lax.rsqrt(jnp.maximum(deg, 1.0))
        isd_ref[...] = isd
        h = jnp.dot(x_ref[...], w_ref[...], preferred_element_type=_f32)
        h2_ref[...] = h * isd[:N_NODES][:, None]

    return pl.pallas_call(
        body,
        out_shape=[
            jax.ShapeDtypeStruct((N_NODES, D), _f32),
            jax.ShapeDtypeStruct((ACCD,), _f32),
        ],
    )(x, W_emb, degp)


def _sc_isdsrc(isd, srcp):
    @functools.partial(
        pl.kernel,
        out_type=jax.ShapeDtypeStruct((NT, NCH, CHUNK), _f32),
        mesh=_mesh(),
        scratch_types=[
            pltpu.VMEM((NCH, CHUNK), _i32),
            pltpu.VMEM((NCH, CHUNK), _f32),
            pltpu.VMEM((ACCD,), _f32),
            pltpu.VMEM_SHARED((ACCD,), _f32),
            pltpu.SemaphoreType.DMA,
        ],
    )
    def body(isd_h, src_h, out_h, srcv, isdall, isdv, isd_s, sem):
        c = lax.axis_index("c")
        s = lax.axis_index("s")
        t = c * NS + s
        pltpu.sync_copy(src_h.at[t], srcv)

        @pl.when(s == 0)
        def _():
            pltpu.sync_copy(isd_h, isdv)
            pltpu.sync_copy(isdv, isd_s)

        plsc.subcore_barrier()
        done = 0
        for b in range((NCH + 7) // 8):
            k = min(8, NCH - done)
            descs = [
                pltpu.async_copy(isd_s.at[srcv.at[done + i]],
                                 isdall.at[done + i], sem)
                for i in range(k)
            ]
            for d in descs:
                d.wait()
            done += k
        pltpu.sync_copy(isdall, out_h.at[t])

    return body(isd, srcp)


def _sc_aggH(h2, srcp, dstp, zH):
    @functools.partial(
        pl.kernel,
        out_type=jax.ShapeDtypeStruct((NC, ACC, D), _f32),
        mesh=_mesh(),
        scratch_types=[
            pltpu.VMEM((NCH, CHUNK), _i32),
            pltpu.VMEM((NCH, CHUNK), _i32),
            pltpu.VMEM((CHUNK, D), _f32),
            pltpu.VMEM_SHARED((ACC, D), _f32),
            pltpu.SemaphoreType.DMA,
        ],
    )
    def body(h2_h, src_h, dst_h, zH_h, aggH_o,
             srcv, dstv, hrows, aggH_s, sem):
        c = lax.axis_index("c")
        s = lax.axis_index("s")
        t = c * NS + s
        pltpu.sync_copy(src_h.at[t], srcv)
        pltpu.sync_copy(dst_h.at[t], dstv)
        pltpu.sync_copy(zH_h, aggH_s.at[pl.ds(s * ZROWS, ZROWS)])
        plsc.subcore_barrier()

        def step(j, carry):
            pltpu.async_copy(h2_h.at[srcv.at[j]], hrows, sem).wait()
            pltpu.sync_copy(hrows, aggH_s.at[dstv.at[j]], add=True)
            return carry

        lax.fori_loop(0, NCH, step, 0)
        plsc.subcore_barrier()
        pltpu.sync_copy(aggH_s.at[pl.ds(s * ZROWS, ZROWS)],
                        aggH_o.at[c, pl.ds(s * ZROWS, ZROWS)])

    return body(h2, srcp, dstp, zH)


def _tc_scale(eap, isdsrc):
    G = 64
    RB = NE_PAD // G

    def body(ea_ref, s_ref, out_ref):
        out_ref[...] = ea_ref[...] * s_ref[...]

    return pl.pallas_call(
        body,
        grid=(G,),
        in_specs=[
            pl.BlockSpec((RB, DE), lambda i: (i, 0)),
            pl.BlockSpec((RB, 1), lambda i: (i, 0)),
        ],
        out_specs=pl.BlockSpec((RB, DE), lambda i: (i, 0)),
        out_shape=jax.ShapeDtypeStruct((NE_PAD, DE), _f32),
    )(eap, isdsrc.reshape(NE_PAD, 1))


def _sc_agg16(ea2p, dstp, z16):
    @functools.partial(
        pl.kernel,
        out_type=jax.ShapeDtypeStruct((NC, ACC, DE), _f32),
        mesh=_mesh(),
        scratch_types=[
            pltpu.VMEM((NCH, CHUNK), _i32),
            pltpu.VMEM((CHUNK, DE), _f32),
            pltpu.VMEM_SHARED((ACC, DE), _f32),
            pltpu.SemaphoreType.DMA,
        ],
    )
    def body(ea2_h, dst_h, z16_h, agg16_o, dstv, eav, agg16_s, sem):
        c = lax.axis_index("c")
        s = lax.axis_index("s")
        t = c * NS + s
        pltpu.sync_copy(dst_h.at[t], dstv)
        pltpu.sync_copy(z16_h, agg16_s.at[pl.ds(s * ZROWS, ZROWS)])
        plsc.subcore_barrier()
        base = t * EPT

        def step(j, carry):
            pltpu.sync_copy(ea2_h.at[pl.ds(base + j * CHUNK, CHUNK)], eav)
            pltpu.sync_copy(eav, agg16_s.at[dstv.at[j]], add=True)
            return carry

        lax.fori_loop(0, NCH, step, 0)
        plsc.subcore_barrier()
        pltpu.sync_copy(agg16_s.at[pl.ds(s * ZROWS, ZROWS)],
                        agg16_o.at[c, pl.ds(s * ZROWS, ZROWS)])

    return body(ea2p, dstp, z16)


def _tc_node(aggHp, agg16p, isd, W_edge, W_gcn, b_gcn):
    def body(aggH_ref, agg16_ref, isd_ref, we_ref, wg_ref, b_ref, out_ref):
        aggH = aggH_ref[0, :N_NODES] + aggH_ref[1, :N_NODES]
        agg16 = agg16_ref[0, :N_NODES] + agg16_ref[1, :N_NODES]
        wg = wg_ref[...]
        wec = jnp.dot(we_ref[...], wg, preferred_element_type=_f32)
        tt = (jnp.dot(aggH, wg, preferred_element_type=_f32) +
              jnp.dot(agg16, wec, preferred_element_type=_f32))
        isd = isd_ref[...][:N_NODES]
        out_ref[...] = jnp.maximum(
            tt * isd[:, None] + b_ref[...][None, :], 0.0)

    return pl.pallas_call(
        body,
        out_shape=jax.ShapeDtypeStruct((N_NODES, D), _f32),
    )(aggHp, agg16p, isd, W_edge, W_gcn, b_gcn)


def _sc_sup(node_emb, sup4):
    @functools.partial(
        pl.kernel,
        out_type=jax.ShapeDtypeStruct((4, N_SUP, D), _f32),
        mesh=_mesh(),
        scratch_types=[
            pltpu.VMEM((SUP_T,), _i32),
            pltpu.VMEM((SUP_T, D), _f32),
            pltpu.SemaphoreType.DMA,
        ],
    )
    def body(ne_h, sup_h, out_h, idxv, rows, sem):
        c = lax.axis_index("c")
        s = lax.axis_index("s")
        t = c * NS + s
        for k in range(4):
            pltpu.sync_copy(sup_h.at[k, t], idxv)
            pltpu.async_copy(ne_h.at[idxv], rows, sem).wait()
            pltpu.sync_copy(rows, out_h.at[k, pl.ds(t * SUP_T, SUP_T)])

    return body(node_emb, sup4)


def _tc_loss(rows4):
    def body(r_ref, out_ref):
        ps = jnp.sum(r_ref[0] * r_ref[1], axis=1)
        ns = jnp.sum(r_ref[2] * r_ref[3], axis=1)

        def sp(v):
            return jnp.maximum(v, 0.0) + jnp.log1p(jnp.exp(-jnp.abs(v)))

        loss = jnp.mean(sp(-ps)) + jnp.mean(sp(ns))
        out_ref[...] = jnp.broadcast_to(loss, (1, D))

    return pl.pallas_call(
        body,
        out_shape=jax.ShapeDtypeStruct((1, D), _f32),
    )(rows4)


def kernel(x, edge_attr, W_emb, W_edge, W_gcn, b_gcn, edge_index,
           pos_sup_edge_index, neg_sup_edge_index):
    src = edge_index[0]
    dst = edge_index[1]
    npad = NE_PAD - N_EDGES
    dstp = jnp.concatenate([dst, jnp.full((npad,), N_NODES, _i32)]).reshape(NT, NCH, CHUNK)
    srcp = jnp.concatenate([src, jnp.zeros((npad,), _i32)]).reshape(NT, NCH, CHUNK)
    eap = jnp.concatenate([edge_attr, jnp.zeros((npad, DE), _f32)])
    sup4 = jnp.concatenate([pos_sup_edge_index, neg_sup_edge_index]).reshape(4, NT, SUP_T)
    ones_v = jnp.ones((CHUNK,), _f32)
    zvec = jnp.zeros((640,), _f32)
    zH = jnp.zeros((ZROWS, D), _f32)
    z16 = jnp.zeros((ZROWS, DE), _f32)

    degp = _sc_deg(dstp, ones_v, zvec)
    h2, isd = _tc_emb(x, W_emb, degp)
    isdsrc = _sc_isdsrc(isd, srcp)
    aggHp = _sc_aggH(h2, srcp, dstp, zH)
    ea2p = _tc_scale(eap, isdsrc.reshape(NE_PAD))
    agg16p = _sc_agg16(ea2p, dstp, z16)
    node_emb = _tc_node(aggHp, agg16p, isd, W_edge, W_gcn, b_gcn)
    rows4 = _sc_sup(node_emb, sup4)
    loss = _tc_loss(rows4)
    return loss[0, 0]
